# Initial kernel scaffold; baseline (speedup 1.0000x reference)
import jax
import jax.numpy as jnp
from jax.experimental import pallas as pl


def kernel(x, weight):
    raise NotImplementedError("write your pallas kernel here")


# speedup vs baseline: 1.1060x; 1.1060x over previous
import functools

import jax
import jax.numpy as jnp
from jax import lax
from jax.experimental import pallas as pl
from jax.experimental.pallas import tpu as pltpu
from jax.experimental.pallas import tpu_sc as plsc

_VOCAB = 1000000
_D = 32
_N = 16384 * 50
_NC = 2
_NS = 16
_NW = _NC * _NS
_PER_W = _N // _NW
_CHUNK = 1280
_NCH = _PER_W // _CHUNK


def _emb_body(w_hbm, idx_hbm, out_hbm, idx_v, rows_v, gsem):
    wid = lax.axis_index("s") * _NC + lax.axis_index("c")
    base = pl.multiple_of(wid * _PER_W, 8)
    pltpu.sync_copy(idx_hbm.at[pl.ds(base, _PER_W)], idx_v)

    def chunk(c, carry):
        off = pl.multiple_of(c * _CHUNK, 8)
        pltpu.async_copy(
            w_hbm.at[idx_v.at[pl.ds(off, _CHUNK)]], rows_v, gsem
        ).wait()
        pltpu.sync_copy(rows_v, out_hbm.at[pl.ds(base + off, _CHUNK)])
        return carry

    lax.fori_loop(0, _NCH, chunk, 0)


@jax.jit
def _emb_lookup(weight, idx_flat):
    mesh = plsc.VectorSubcoreMesh(core_axis_name="c", subcore_axis_name="s")
    return pl.kernel(
        _emb_body,
        out_type=jax.ShapeDtypeStruct((_N, _D), jnp.float32),
        mesh=mesh,
        compiler_params=pltpu.CompilerParams(use_tc_tiling_on_sc=False),
        scratch_types=[
            pltpu.VMEM((_PER_W,), jnp.int32),
            pltpu.VMEM((_CHUNK, _D), jnp.float32),
            pltpu.SemaphoreType.DMA,
        ],
    )(weight, idx_flat)


def kernel(x, weight):
    idx_flat = x.reshape(-1).astype(jnp.int32)
    out = _emb_lookup(weight, idx_flat)
    return out.reshape(x.shape + (weight.shape[1],))


# speedup vs baseline: 1.6494x; 1.4913x over previous
import functools

import jax
import jax.numpy as jnp
from jax import lax
from jax.experimental import pallas as pl
from jax.experimental.pallas import tpu as pltpu
from jax.experimental.pallas import tpu_sc as plsc

_VOCAB = 1000000
_D = 32
_B = 16384
_H = 50
_N = _B * _H
_NC = 2
_NS = 16
_NW = _NC * _NS
_U = 512
_UNITS = _N // _U
_UPW = _UNITS // _NW
_GPH = _B // _U


def _transpose_unit(src_v, tb, iota16):
    def lg_body(lg, carry):
        rows = lg * 16 + iota16
        base = ((lg // 8) * 8) * 128 + (lg % 8) * 16
        for f in range(_D):
            cols = jnp.full((16,), f, dtype=jnp.int32)
            vec = plsc.load_gather(src_v, [rows, cols])
            off = (f // 8) * 4096 + base + (f % 8) * 128
            tb[pl.ds(pl.multiple_of(off, 8), 16)] = vec
        return carry

    lax.fori_loop(0, _U // 16, lg_body, 0)


def _emb_body(w_hbm, xcm_hbm, out_hbm, idx_v, src0, src1, tb0, tb1,
              g0, g1, w0, w1):
    wid = lax.axis_index("s") * _NC + lax.axis_index("c")
    u0 = wid * _UPW
    iota16 = lax.iota(jnp.int32, 16)

    pltpu.sync_copy(
        xcm_hbm.at[pl.ds(pl.multiple_of(u0 * _U, 8), _UPW * _U)], idx_v)

    def start_gather(uu_local, src, gsem):
        off = pl.multiple_of(uu_local * _U, 8)
        pltpu.async_copy(w_hbm.at[idx_v.at[pl.ds(off, _U)]], src, gsem)

    def unit_out_base(uu, fg):
        h = uu // _GPH
        g = uu % _GPH
        return pl.multiple_of(((h * 4 + fg) * 128 + 4 * g) * 1024, 8)

    def start_writes(uu, tb, wsem):
        for fg in range(4):
            pltpu.async_copy(
                tb.at[pl.ds(fg * 4096, 4096)],
                out_hbm.at[pl.ds(unit_out_base(uu, fg), 4096)],
                wsem)

    def wait_writes(uu, tb, wsem):
        for fg in range(4):
            pltpu.make_async_copy(
                tb.at[pl.ds(fg * 4096, 4096)],
                out_hbm.at[pl.ds(unit_out_base(uu, fg), 4096)],
                wsem).wait()

    start_gather(0, src0, g0)

    def pair_body(i, carry):
        for k, (src, tb, gsem, wsem) in enumerate(
                ((src0, tb0, g0, w0), (src1, tb1, g1, w1))):
            ul = 2 * i + k
            uu = u0 + ul

            @pl.when(ul + 1 < _UPW)
            def _():
                start_gather(ul + 1, src1 if k == 0 else src0,
                             g1 if k == 0 else g0)

            pltpu.make_async_copy(
                w_hbm.at[idx_v.at[pl.ds(pl.multiple_of(ul * _U, 8), _U)]],
                src, gsem).wait()

            @pl.when(ul >= 2)
            def _():
                wait_writes(uu - 2, tb, wsem)

            _transpose_unit(src, tb, iota16)
            start_writes(uu, tb, wsem)
        return carry

    lax.fori_loop(0, _UPW // 2, pair_body, 0)

    wait_writes(u0 + _UPW - 2, tb0, w0)
    wait_writes(u0 + _UPW - 1, tb1, w1)


@jax.jit
def _emb_lookup(weight, xcm):
    mesh = plsc.VectorSubcoreMesh(core_axis_name="c", subcore_axis_name="s")
    return pl.kernel(
        _emb_body,
        out_type=jax.ShapeDtypeStruct((_N * _D,), jnp.float32),
        mesh=mesh,
        compiler_params=pltpu.CompilerParams(
            use_tc_tiling_on_sc=False, needs_layout_passes=False),
        scratch_types=[
            pltpu.VMEM((_UPW * _U,), jnp.int32),
            pltpu.VMEM((_U, _D), jnp.float32),
            pltpu.VMEM((_U, _D), jnp.float32),
            pltpu.VMEM((_U * _D,), jnp.float32),
            pltpu.VMEM((_U * _D,), jnp.float32),
            pltpu.SemaphoreType.DMA,
            pltpu.SemaphoreType.DMA,
            pltpu.SemaphoreType.DMA,
            pltpu.SemaphoreType.DMA,
        ],
    )(weight, xcm)


def kernel(x, weight):
    xcm = x.astype(jnp.int32).T.reshape(-1)
    out1d = _emb_lookup(weight, xcm)
    ko = out1d.reshape(_H, 4, _B // 128, 8, 128)
    return ko.transpose(2, 4, 0, 1, 3).reshape(_B, _H, _D)


# speedup vs baseline: 2.1389x; 1.2967x over previous
import functools

import jax
import jax.numpy as jnp
from jax import lax
from jax.experimental import pallas as pl
from jax.experimental.pallas import tpu as pltpu
from jax.experimental.pallas import tpu_sc as plsc

_VOCAB = 1000000
_D = 32
_B = 16384
_H = 50
_N = _B * _H
_NC = 2
_NS = 16
_NW = _NC * _NS
_U = 512
_UNITS = _N // _U
_UPW = _UNITS // _NW
_GPH = _B // _U


def _transpose_unit(src_v, tb, iota16):
    def _lg_body(lg, carry):
        rows = (lg << 4) + iota16
        base = ((lg >> 3) << 10) + ((lg & 7) << 4)
        for half in range(2):
            vecs = [
                plsc.load_gather(
                    src_v,
                    [rows, jnp.full((16,), half * 16 + s, dtype=jnp.int32)])
                for s in range(16)
            ]
            for s in range(16):
                f = half * 16 + s
                off = base + (f // 8) * 4096 + (f % 8) * 128
                tb[pl.ds(pl.multiple_of(off, 8), 16)] = vecs[s]
        return carry

    lax.fori_loop(0, _U // 16, _lg_body, 0)


def _emb_body(w_hbm, xcm_hbm, out_hbm, idx_v, src0, src1, tb0, tb1,
              g0, g1, w0, w1):
    wid = lax.axis_index("s") * _NC + lax.axis_index("c")
    u0 = wid * _UPW
    iota16 = lax.iota(jnp.int32, 16)

    pltpu.sync_copy(
        xcm_hbm.at[pl.ds(pl.multiple_of(u0 * _U, 8), _UPW * _U)], idx_v)

    def start_gather(uu_local, src, gsem):
        off = pl.multiple_of(uu_local * _U, 8)
        pltpu.async_copy(w_hbm.at[idx_v.at[pl.ds(off, _U)]], src, gsem)

    def unit_out_base(uu, fg):
        h = uu // _GPH
        g = uu % _GPH
        return pl.multiple_of(((h * 4 + fg) * 128 + 4 * g) * 1024, 8)

    def start_writes(uu, tb, wsem):
        for fg in range(4):
            pltpu.async_copy(
                tb.at[pl.ds(fg * 4096, 4096)],
                out_hbm.at[pl.ds(unit_out_base(uu, fg), 4096)],
                wsem)

    def wait_writes(uu, tb, wsem):
        for fg in range(4):
            pltpu.make_async_copy(
                tb.at[pl.ds(fg * 4096, 4096)],
                out_hbm.at[pl.ds(unit_out_base(uu, fg), 4096)],
                wsem).wait()

    start_gather(0, src0, g0)

    def pair_body(i, carry):
        for k, (src, tb, gsem, wsem) in enumerate(
                ((src0, tb0, g0, w0), (src1, tb1, g1, w1))):
            ul = 2 * i + k
            uu = u0 + ul

            @pl.when(ul + 1 < _UPW)
            def _():
                start_gather(ul + 1, src1 if k == 0 else src0,
                             g1 if k == 0 else g0)

            pltpu.make_async_copy(
                w_hbm.at[idx_v.at[pl.ds(pl.multiple_of(ul * _U, 8), _U)]],
                src, gsem).wait()

            @pl.when(ul >= 2)
            def _():
                wait_writes(uu - 2, tb, wsem)

            _transpose_unit(src, tb, iota16)
            start_writes(uu, tb, wsem)
        return carry

    lax.fori_loop(0, _UPW // 2, pair_body, 0)

    wait_writes(u0 + _UPW - 2, tb0, w0)
    wait_writes(u0 + _UPW - 1, tb1, w1)


@jax.jit
def _emb_lookup(weight, xcm):
    mesh = plsc.VectorSubcoreMesh(core_axis_name="c", subcore_axis_name="s")
    return pl.kernel(
        _emb_body,
        out_type=jax.ShapeDtypeStruct((_N * _D,), jnp.float32),
        mesh=mesh,
        compiler_params=pltpu.CompilerParams(
            use_tc_tiling_on_sc=False, needs_layout_passes=False),
        scratch_types=[
            pltpu.VMEM((_UPW * _U,), jnp.int32),
            pltpu.VMEM((_U, _D), jnp.float32),
            pltpu.VMEM((_U, _D), jnp.float32),
            pltpu.VMEM((_U * _D,), jnp.float32),
            pltpu.VMEM((_U * _D,), jnp.float32),
            pltpu.SemaphoreType.DMA,
            pltpu.SemaphoreType.DMA,
            pltpu.SemaphoreType.DMA,
            pltpu.SemaphoreType.DMA,
        ],
    )(weight, xcm)


def kernel(x, weight):
    xcm = x.astype(jnp.int32).T.reshape(-1)
    out1d = _emb_lookup(weight, xcm)
    ko = out1d.reshape(_H, 4, _B // 128, 8, 128)
    return ko.transpose(2, 4, 0, 1, 3).reshape(_B, _H, _D)
